# Initial kernel scaffold; baseline (speedup 1.0000x reference)
import jax
import jax.numpy as jnp
from jax.experimental import pallas as pl


def kernel(x, W1, b1, g1, be1, W2, b2, g2, be2, W3, b3, g3, be3, W4, b4, g4, be4, W5, b5, g5, be5, W6, b6, W7, b7):
    raise NotImplementedError("write your pallas kernel here")


# speedup vs baseline: 7.0304x; 7.0304x over previous
import functools

import numpy as np
import jax
import jax.numpy as jnp
from jax import lax
from jax.experimental import pallas as pl
from jax.experimental.pallas import tpu as pltpu
from jax.experimental.pallas import tpu_sc as plsc

K = 20
IDXW = 32
INV_SQRT = np.float32(np.sqrt(np.float32(1.0) + np.float32(1e-5)))
NC, NS = 2, 16
NW = NC * NS


def _topk_body(nrows, xrow_ref, xcol_ref, idx_ref):
    b = pl.program_id(0)
    rows_f = xrow_ref[...]
    cols_f = xcol_ref[0]
    R = rows_f.shape[0]
    N = cols_f.shape[1]

    g = jax.lax.dot_general(
        rows_f.astype(jnp.bfloat16), cols_f.astype(jnp.bfloat16),
        (((1,), (0,)), ((), ())), preferred_element_type=jnp.float32)
    inner = -2.0 * g
    xx_rows = jnp.sum(rows_f * rows_f, axis=1, keepdims=True)
    xx_cols = jnp.sum(cols_f * cols_f, axis=0, keepdims=True)
    pd = -xx_rows - inner - xx_cols

    lane = lax.broadcasted_iota(jnp.int32, (R, N), 1)
    colw = lax.broadcasted_iota(jnp.int32, (R, IDXW), 1)
    off = b * nrows

    def body(t, carry):
        pd_c, acc = carry
        m = jnp.max(pd_c, axis=1, keepdims=True)
        cand = jnp.where(pd_c == m, lane, N)
        amax = jnp.min(cand, axis=1, keepdims=True)
        pd_c = jnp.where(lane == amax, -jnp.inf, pd_c)
        acc = jnp.where(colw == t, amax + off, acc)
        return pd_c, acc

    acc0 = jnp.zeros((R, IDXW), jnp.int32)
    _, acc = lax.fori_loop(0, K, body, (pd, acc0))
    idx_ref[...] = acc


def _topk(xrow, xcol, R=256):
    BN, Cp = xrow.shape
    B, _, N = xcol.shape
    nrows = N
    grid = (B, N // R)
    return pl.pallas_call(
        functools.partial(_topk_body, nrows),
        grid=grid,
        in_specs=[
            pl.BlockSpec((R, Cp), lambda b, r: (b * (N // R) + r, 0)),
            pl.BlockSpec((1, Cp, N), lambda b, r: (b, 0, 0)),
        ],
        out_specs=pl.BlockSpec((R, IDXW), lambda b, r: (b * (N // R) + r, 0)),
        out_shape=jax.ShapeDtypeStruct((BN, IDXW), jnp.int32),
    )(xrow, xcol)


def _make_gather(BN, Cp, CH=128):
    nidx = BN * K
    per_w = nidx // NW
    nchunk = per_w // CH
    mesh = plsc.VectorSubcoreMesh(core_axis_name="c", subcore_axis_name="s")

    @functools.partial(
        pl.kernel, mesh=mesh,
        out_type=jax.ShapeDtypeStruct((nidx, Cp), jnp.float32),
        scratch_types=[
            pltpu.VMEM((CH,), jnp.int32),
            pltpu.VMEM((CH, Cp), jnp.float32),
            pltpu.SemaphoreType.DMA,
        ],
    )
    def gather(xrow_hbm, idx_hbm, nb_hbm, idx_v, rows_v, sem):
        wid = lax.axis_index("s") * NC + lax.axis_index("c")
        base_w = wid * per_w

        def chunk(g, _):
            base = base_w + g * CH
            pltpu.sync_copy(idx_hbm.at[pl.ds(base, CH)], idx_v)
            pltpu.async_copy(xrow_hbm.at[idx_v], rows_v, sem).wait()
            pltpu.sync_copy(rows_v, nb_hbm.at[pl.ds(base, CH)])
            return _

        lax.fori_loop(0, nchunk, chunk, 0)

    return gather


def _conv_body(nb_ref, xc_ref, wa_ref, wb_ref, b_ref, g_ref, be_ref, out_ref):
    xc = xc_ref[...]
    R2, Cp = xc.shape
    O = wa_ref.shape[1]
    nb = nb_ref[...]
    xc_rep = jnp.broadcast_to(xc[:, None, :], (R2, K, Cp)).reshape(R2 * K, Cp)
    d = (nb - xc_rep).astype(jnp.bfloat16)
    ya = jax.lax.dot_general(d, wa_ref[...], (((1,), (0,)), ((), ())),
                             preferred_element_type=jnp.float32)
    ya3 = ya.reshape(R2, K, O)
    m = jnp.max(ya3, axis=1)
    yb = jax.lax.dot_general(xc.astype(jnp.bfloat16), wb_ref[...],
                             (((1,), (0,)), ((), ())),
                             preferred_element_type=jnp.float32)
    y = (m + yb) + b_ref[...]
    y = y / INV_SQRT * g_ref[...] + be_ref[...]
    out_ref[...] = jnp.where(y > 0, y, 0.2 * y)


def _conv(nb, xrow, waT, wbT, bvec, gvec, bevec, R2=128):
    BN, Cp = xrow.shape
    O = waT.shape[1]
    grid = (BN // R2,)
    return pl.pallas_call(
        _conv_body,
        grid=grid,
        in_specs=[
            pl.BlockSpec((R2 * K, Cp), lambda r: (r, 0)),
            pl.BlockSpec((R2, Cp), lambda r: (r, 0)),
            pl.BlockSpec((Cp, O), lambda r: (0, 0)),
            pl.BlockSpec((Cp, O), lambda r: (0, 0)),
            pl.BlockSpec((1, O), lambda r: (0, 0)),
            pl.BlockSpec((1, O), lambda r: (0, 0)),
            pl.BlockSpec((1, O), lambda r: (0, 0)),
        ],
        out_specs=pl.BlockSpec((R2, O), lambda r: (r, 0)),
        out_shape=jax.ShapeDtypeStruct((BN, O), jnp.float32),
    )(nb, xrow, waT, wbT, bvec, gvec, bevec)


def _head_body(x1_ref, x2_ref, x3_ref, x4_ref, w5_refs, b5_ref, g5_ref,
               be5_ref, w6_ref, b6_ref, w7_ref, b7_ref, out_ref):
    w51, w52, w53, w54 = w5_refs
    y5 = jax.lax.dot_general(x1_ref[...].astype(jnp.bfloat16), w51[...],
                             (((1,), (0,)), ((), ())),
                             preferred_element_type=jnp.float32)
    y5 += jax.lax.dot_general(x2_ref[...].astype(jnp.bfloat16), w52[...],
                              (((1,), (0,)), ((), ())),
                              preferred_element_type=jnp.float32)
    y5 += jax.lax.dot_general(x3_ref[...].astype(jnp.bfloat16), w53[...],
                              (((1,), (0,)), ((), ())),
                              preferred_element_type=jnp.float32)
    y5 += jax.lax.dot_general(x4_ref[...].astype(jnp.bfloat16), w54[...],
                              (((1,), (0,)), ((), ())),
                              preferred_element_type=jnp.float32)
    y5 = y5 + b5_ref[...]
    y5 = y5 / INV_SQRT * g5_ref[...] + be5_ref[...]
    y5 = jnp.where(y5 > 0, y5, 0.2 * y5)
    y6 = jax.lax.dot_general(y5.astype(jnp.bfloat16), w6_ref[...],
                             (((1,), (0,)), ((), ())),
                             preferred_element_type=jnp.float32)
    y6 = y6 + b6_ref[...]
    y7 = jax.lax.dot_general(y6.astype(jnp.bfloat16), w7_ref[...],
                             (((1,), (0,)), ((), ())),
                             preferred_element_type=jnp.float32)
    out_ref[...] = y7 + b7_ref[...]


def _head(x1, x2, x3, x4, w5Ts, b5, g5, be5, w6T, b6, w7T, b7, R3=512):
    BN = x1.shape[0]
    grid = (BN // R3,)
    O7 = w7T.shape[1]

    def body(x1r, x2r, x3r, x4r, w51, w52, w53, w54, b5r, g5r, be5r,
             w6r, b6r, w7r, b7r, outr):
        _head_body(x1r, x2r, x3r, x4r, (w51, w52, w53, w54), b5r, g5r,
                   be5r, w6r, b6r, w7r, b7r, outr)

    full = lambda a: pl.BlockSpec(a.shape, lambda r: tuple(0 for _ in a.shape))
    row = lambda a: pl.BlockSpec((R3, a.shape[1]), lambda r: (r, 0))
    args = (x1, x2, x3, x4, *w5Ts, b5, g5, be5, w6T, b6, w7T, b7)
    specs = [row(x1), row(x2), row(x3), row(x4)] + [full(a) for a in args[4:]]
    return pl.pallas_call(
        body,
        grid=grid,
        in_specs=specs,
        out_specs=pl.BlockSpec((R3, O7), lambda r: (r, 0)),
        out_shape=jax.ShapeDtypeStruct((BN, O7), jnp.float32),
    )(*args)


def _pad_cols(a, w):
    if a.shape[-1] == w:
        return a
    return jnp.zeros(a.shape[:-1] + (w,), a.dtype).at[..., :a.shape[-1]].set(a)


def _edge_layer(xrow, xcol, W, b, g, be, C, Cp, O, Op):
    BN = xrow.shape[0]
    idx = _topk(xrow, xcol)[:, :K].reshape(-1)
    nb = _make_gather(BN, Cp)(xrow, idx)
    waT = jnp.zeros((Cp, Op), jnp.bfloat16).at[:C, :O].set(
        W[:, :C].T.astype(jnp.bfloat16))
    wbT = jnp.zeros((Cp, Op), jnp.bfloat16).at[:C, :O].set(
        W[:, C:].T.astype(jnp.bfloat16))
    return _conv(nb, xrow, waT, wbT, _pad_cols(b[None, :], Op),
                 _pad_cols(g[None, :], Op), _pad_cols(be[None, :], Op))


def kernel(x, W1, b1, g1, be1, W2, b2, g2, be2, W3, b3, g3, be3,
           W4, b4, g4, be4, W5, b5, g5, be5, W6, b6, W7, b7):
    B, C0, N = x.shape
    BN = B * N
    xr = jnp.transpose(x, (0, 2, 1)).reshape(BN, C0)
    xrow1 = jnp.zeros((BN, 128), jnp.float32).at[:, :C0].set(xr)
    xcol1 = jnp.zeros((B, 128, N), jnp.float32).at[:, :C0].set(x)

    x1 = _edge_layer(xrow1, xcol1, W1, b1, g1, be1, C0, 128, 64, 128)
    xcol2 = jnp.transpose(x1.reshape(B, N, 128), (0, 2, 1))
    x2 = _edge_layer(x1, xcol2, W2, b2, g2, be2, 64, 128, 64, 128)
    xcol3 = jnp.transpose(x2.reshape(B, N, 128), (0, 2, 1))
    x3 = _edge_layer(x2, xcol3, W3, b3, g3, be3, 64, 128, 128, 128)
    xcol4 = jnp.transpose(x3.reshape(B, N, 128), (0, 2, 1))
    x4 = _edge_layer(x3, xcol4, W4, b4, g4, be4, 128, 128, 256, 256)

    w5Ts = tuple(
        jnp.zeros((128, 1024), jnp.bfloat16).at[:hi - lo].set(
            W5[:, lo:hi].T.astype(jnp.bfloat16))
        for lo, hi in ((0, 64), (64, 128), (128, 256)))
    w5Ts = w5Ts + (W5[:, 256:512].T.astype(jnp.bfloat16),)
    w7T = jnp.zeros((256, 16), jnp.bfloat16).at[:, :13].set(
        W7.T.astype(jnp.bfloat16))
    b7p = jnp.zeros((1, 16), jnp.float32).at[:, :13].set(b7[None, :])
    out = _head(x1, x2, x3, x4, w5Ts, b5[None, :], g5[None, :], be5[None, :],
                W6.T.astype(jnp.bfloat16), b6[None, :], w7T, b7p)
    out = out.reshape(B, N, 16)[:, :, :13]
    return jnp.transpose(out, (0, 2, 1))


# speedup vs baseline: 7.3790x; 1.0496x over previous
import functools

import numpy as np
import jax
import jax.numpy as jnp
from jax import lax
from jax.experimental import pallas as pl
from jax.experimental.pallas import tpu as pltpu
from jax.experimental.pallas import tpu_sc as plsc

K = 20
IDXW = 32
INV_SQRT = np.float32(np.sqrt(np.float32(1.0) + np.float32(1e-5)))
NC, NS = 2, 16
NW = NC * NS


def _topk_body(nrows, xrow_ref, xcol_ref, idx_ref):
    b = pl.program_id(0)
    rows_f = xrow_ref[...]
    cols_f = xcol_ref[0]
    R = rows_f.shape[0]
    N = cols_f.shape[1]

    g = jax.lax.dot_general(
        rows_f.astype(jnp.bfloat16), cols_f.astype(jnp.bfloat16),
        (((1,), (0,)), ((), ())), preferred_element_type=jnp.float32)
    inner = -2.0 * g
    xx_rows = jnp.sum(rows_f * rows_f, axis=1, keepdims=True)
    xx_cols = jnp.sum(cols_f * cols_f, axis=0, keepdims=True)
    pd = -xx_rows - inner - xx_cols

    lane = lax.broadcasted_iota(jnp.int32, (R, N), 1)
    colw = lax.broadcasted_iota(jnp.int32, (R, IDXW), 1)
    off = b * nrows

    def body(t, carry):
        pd_c, acc = carry
        m = jnp.max(pd_c, axis=1, keepdims=True)
        cand = jnp.where(pd_c == m, lane, N)
        amax = jnp.min(cand, axis=1, keepdims=True)
        pd_c = jnp.where(lane == amax, -jnp.inf, pd_c)
        acc = jnp.where(colw == t, amax + off, acc)
        return pd_c, acc

    acc0 = jnp.zeros((R, IDXW), jnp.int32)
    _, acc = lax.fori_loop(0, K, body, (pd, acc0))
    idx_ref[...] = acc


def _topk(xrow, xcol, R=256):
    BN, Cp = xrow.shape
    B, _, N = xcol.shape
    nrows = N
    grid = (B, N // R)
    return pl.pallas_call(
        functools.partial(_topk_body, nrows),
        grid=grid,
        in_specs=[
            pl.BlockSpec((R, Cp), lambda b, r: (b * (N // R) + r, 0)),
            pl.BlockSpec((1, Cp, N), lambda b, r: (b, 0, 0)),
        ],
        out_specs=pl.BlockSpec((R, IDXW), lambda b, r: (b * (N // R) + r, 0)),
        out_shape=jax.ShapeDtypeStruct((BN, IDXW), jnp.int32),
    )(xrow, xcol)


def _make_gather(BN, Cp, CH=128):
    nidx = BN * K
    per_w = nidx // NW
    npair = per_w // (2 * CH)
    mesh = plsc.VectorSubcoreMesh(core_axis_name="c", subcore_axis_name="s")

    @functools.partial(
        pl.kernel, mesh=mesh,
        out_type=jax.ShapeDtypeStruct((nidx, Cp), jnp.float32),
        scratch_types=[
            pltpu.VMEM((2, CH), jnp.int32),
            pltpu.VMEM((2, CH, Cp), jnp.float32),
            pltpu.SemaphoreType.DMA,
            pltpu.SemaphoreType.DMA,
            pltpu.SemaphoreType.DMA,
            pltpu.SemaphoreType.DMA,
        ],
    )
    def gather(xrow_hbm, idx_hbm, nb_hbm, idx2, rows2, sg0, sg1, sw0, sw1):
        wid = lax.axis_index("s") * NC + lax.axis_index("c")
        base_w = wid * per_w
        sg = (sg0, sg1)
        sw = (sw0, sw1)

        def pair(p, _):
            hs = []
            for b in (0, 1):
                base = base_w + (2 * p + b) * CH

                @pl.when(p > 0)
                def _drain(b=b, base=base):
                    pltpu.make_async_copy(
                        rows2.at[b], nb_hbm.at[pl.ds(base, CH)], sw[b]).wait()

                pltpu.sync_copy(idx_hbm.at[pl.ds(base, CH)], idx2.at[b])
                hs.append(pltpu.async_copy(
                    xrow_hbm.at[idx2.at[b]], rows2.at[b], sg[b]))
            for b in (0, 1):
                base = base_w + (2 * p + b) * CH
                hs[b].wait()
                pltpu.async_copy(rows2.at[b], nb_hbm.at[pl.ds(base, CH)], sw[b])
            return _

        lax.fori_loop(0, npair, pair, 0)
        for b in (0, 1):
            pltpu.make_async_copy(
                rows2.at[b], nb_hbm.at[pl.ds(base_w, CH)], sw[b]).wait()

    return gather


def _conv_body(nb_ref, xc_ref, wa_ref, wb_ref, b_ref, g_ref, be_ref, out_ref):
    xc = xc_ref[...]
    R2, Cp = xc.shape
    O = wa_ref.shape[1]
    nb = nb_ref[...]
    xc_rep = jnp.broadcast_to(xc[:, None, :], (R2, K, Cp)).reshape(R2 * K, Cp)
    d = (nb - xc_rep).astype(jnp.bfloat16)
    ya = jax.lax.dot_general(d, wa_ref[...], (((1,), (0,)), ((), ())),
                             preferred_element_type=jnp.float32)
    ya3 = ya.reshape(R2, K, O)
    m = jnp.max(ya3, axis=1)
    yb = jax.lax.dot_general(xc.astype(jnp.bfloat16), wb_ref[...],
                             (((1,), (0,)), ((), ())),
                             preferred_element_type=jnp.float32)
    y = (m + yb) + b_ref[...]
    y = y / INV_SQRT * g_ref[...] + be_ref[...]
    out_ref[...] = jnp.where(y > 0, y, 0.2 * y)


def _conv(nb, xrow, waT, wbT, bvec, gvec, bevec, R2=128):
    BN, Cp = xrow.shape
    O = waT.shape[1]
    grid = (BN // R2,)
    return pl.pallas_call(
        _conv_body,
        grid=grid,
        in_specs=[
            pl.BlockSpec((R2 * K, Cp), lambda r: (r, 0)),
            pl.BlockSpec((R2, Cp), lambda r: (r, 0)),
            pl.BlockSpec((Cp, O), lambda r: (0, 0)),
            pl.BlockSpec((Cp, O), lambda r: (0, 0)),
            pl.BlockSpec((1, O), lambda r: (0, 0)),
            pl.BlockSpec((1, O), lambda r: (0, 0)),
            pl.BlockSpec((1, O), lambda r: (0, 0)),
        ],
        out_specs=pl.BlockSpec((R2, O), lambda r: (r, 0)),
        out_shape=jax.ShapeDtypeStruct((BN, O), jnp.float32),
    )(nb, xrow, waT, wbT, bvec, gvec, bevec)


def _head_body(x1_ref, x2_ref, x3_ref, x4_ref, w5_refs, b5_ref, g5_ref,
               be5_ref, w6_ref, b6_ref, w7_ref, b7_ref, out_ref):
    w51, w52, w53, w54 = w5_refs
    y5 = jax.lax.dot_general(x1_ref[...].astype(jnp.bfloat16), w51[...],
                             (((1,), (0,)), ((), ())),
                             preferred_element_type=jnp.float32)
    y5 += jax.lax.dot_general(x2_ref[...].astype(jnp.bfloat16), w52[...],
                              (((1,), (0,)), ((), ())),
                              preferred_element_type=jnp.float32)
    y5 += jax.lax.dot_general(x3_ref[...].astype(jnp.bfloat16), w53[...],
                              (((1,), (0,)), ((), ())),
                              preferred_element_type=jnp.float32)
    y5 += jax.lax.dot_general(x4_ref[...].astype(jnp.bfloat16), w54[...],
                              (((1,), (0,)), ((), ())),
                              preferred_element_type=jnp.float32)
    y5 = y5 + b5_ref[...]
    y5 = y5 / INV_SQRT * g5_ref[...] + be5_ref[...]
    y5 = jnp.where(y5 > 0, y5, 0.2 * y5)
    y6 = jax.lax.dot_general(y5.astype(jnp.bfloat16), w6_ref[...],
                             (((1,), (0,)), ((), ())),
                             preferred_element_type=jnp.float32)
    y6 = y6 + b6_ref[...]
    y7 = jax.lax.dot_general(y6.astype(jnp.bfloat16), w7_ref[...],
                             (((1,), (0,)), ((), ())),
                             preferred_element_type=jnp.float32)
    out_ref[...] = y7 + b7_ref[...]


def _head(x1, x2, x3, x4, w5Ts, b5, g5, be5, w6T, b6, w7T, b7, R3=512):
    BN = x1.shape[0]
    grid = (BN // R3,)
    O7 = w7T.shape[1]

    def body(x1r, x2r, x3r, x4r, w51, w52, w53, w54, b5r, g5r, be5r,
             w6r, b6r, w7r, b7r, outr):
        _head_body(x1r, x2r, x3r, x4r, (w51, w52, w53, w54), b5r, g5r,
                   be5r, w6r, b6r, w7r, b7r, outr)

    full = lambda a: pl.BlockSpec(a.shape, lambda r: tuple(0 for _ in a.shape))
    row = lambda a: pl.BlockSpec((R3, a.shape[1]), lambda r: (r, 0))
    args = (x1, x2, x3, x4, *w5Ts, b5, g5, be5, w6T, b6, w7T, b7)
    specs = [row(x1), row(x2), row(x3), row(x4)] + [full(a) for a in args[4:]]
    return pl.pallas_call(
        body,
        grid=grid,
        in_specs=specs,
        out_specs=pl.BlockSpec((R3, O7), lambda r: (r, 0)),
        out_shape=jax.ShapeDtypeStruct((BN, O7), jnp.float32),
    )(*args)


def _pad_cols(a, w):
    if a.shape[-1] == w:
        return a
    return jnp.zeros(a.shape[:-1] + (w,), a.dtype).at[..., :a.shape[-1]].set(a)


def _edge_layer(xrow, xcol, W, b, g, be, C, Cp, O, Op):
    BN = xrow.shape[0]
    idx = _topk(xrow, xcol)[:, :K].reshape(-1)
    nb = _make_gather(BN, Cp)(xrow, idx)
    waT = jnp.zeros((Cp, Op), jnp.bfloat16).at[:C, :O].set(
        W[:, :C].T.astype(jnp.bfloat16))
    wbT = jnp.zeros((Cp, Op), jnp.bfloat16).at[:C, :O].set(
        W[:, C:].T.astype(jnp.bfloat16))
    return _conv(nb, xrow, waT, wbT, _pad_cols(b[None, :], Op),
                 _pad_cols(g[None, :], Op), _pad_cols(be[None, :], Op))


def kernel(x, W1, b1, g1, be1, W2, b2, g2, be2, W3, b3, g3, be3,
           W4, b4, g4, be4, W5, b5, g5, be5, W6, b6, W7, b7):
    B, C0, N = x.shape
    BN = B * N
    xr = jnp.transpose(x, (0, 2, 1)).reshape(BN, C0)
    xrow1 = jnp.zeros((BN, 128), jnp.float32).at[:, :C0].set(xr)
    xcol1 = jnp.zeros((B, 128, N), jnp.float32).at[:, :C0].set(x)

    x1 = _edge_layer(xrow1, xcol1, W1, b1, g1, be1, C0, 128, 64, 128)
    xcol2 = jnp.transpose(x1.reshape(B, N, 128), (0, 2, 1))
    x2 = _edge_layer(x1, xcol2, W2, b2, g2, be2, 64, 128, 64, 128)
    xcol3 = jnp.transpose(x2.reshape(B, N, 128), (0, 2, 1))
    x3 = _edge_layer(x2, xcol3, W3, b3, g3, be3, 64, 128, 128, 128)
    xcol4 = jnp.transpose(x3.reshape(B, N, 128), (0, 2, 1))
    x4 = _edge_layer(x3, xcol4, W4, b4, g4, be4, 128, 128, 256, 256)

    w5Ts = tuple(
        jnp.zeros((128, 1024), jnp.bfloat16).at[:hi - lo].set(
            W5[:, lo:hi].T.astype(jnp.bfloat16))
        for lo, hi in ((0, 64), (64, 128), (128, 256)))
    w5Ts = w5Ts + (W5[:, 256:512].T.astype(jnp.bfloat16),)
    w7T = jnp.zeros((256, 16), jnp.bfloat16).at[:, :13].set(
        W7.T.astype(jnp.bfloat16))
    b7p = jnp.zeros((1, 16), jnp.float32).at[:, :13].set(b7[None, :])
    out = _head(x1, x2, x3, x4, w5Ts, b5[None, :], g5[None, :], be5[None, :],
                W6.T.astype(jnp.bfloat16), b6[None, :], w7T, b7p)
    out = out.reshape(B, N, 16)[:, :, :13]
    return jnp.transpose(out, (0, 2, 1))


# speedup vs baseline: 7.7328x; 1.0480x over previous
import functools

import numpy as np
import jax
import jax.numpy as jnp
from jax import lax
from jax.experimental import pallas as pl
from jax.experimental.pallas import tpu as pltpu
from jax.experimental.pallas import tpu_sc as plsc

K = 20
IDXW = 32
INV_SQRT = np.float32(np.sqrt(np.float32(1.0) + np.float32(1e-5)))
NC, NS = 2, 16
NW = NC * NS


def _topk_body(nrows, boff, xrow_ref, xcol_ref, idx_ref):
    b = pl.program_id(0) + boff
    rows_f = xrow_ref[...]
    cols_f = xcol_ref[0]
    R = rows_f.shape[0]
    N = cols_f.shape[1]

    g = jax.lax.dot_general(
        rows_f.astype(jnp.bfloat16), cols_f.astype(jnp.bfloat16),
        (((1,), (0,)), ((), ())), preferred_element_type=jnp.float32)
    inner = -2.0 * g
    xx_rows = jnp.sum(rows_f * rows_f, axis=1, keepdims=True)
    xx_cols = jnp.sum(cols_f * cols_f, axis=0, keepdims=True)
    pd = -xx_rows - inner - xx_cols

    lane = lax.broadcasted_iota(jnp.int32, (R, N), 1)
    colw = lax.broadcasted_iota(jnp.int32, (R, IDXW), 1)
    off = b * nrows

    def body(t, carry):
        pd_c, acc = carry
        m = jnp.max(pd_c, axis=1, keepdims=True)
        cand = jnp.where(pd_c == m, lane, N)
        amax = jnp.min(cand, axis=1, keepdims=True)
        pd_c = jnp.where(lane == amax, -jnp.inf, pd_c)
        acc = jnp.where(colw == t, amax + off, acc)
        return pd_c, acc

    acc0 = jnp.zeros((R, IDXW), jnp.int32)
    _, acc = lax.fori_loop(0, K, body, (pd, acc0))
    idx_ref[...] = acc


def _topk(xrow, xcol, boff=0, R=256):
    BN, Cp = xrow.shape
    B, _, N = xcol.shape
    nrows = N
    grid = (B, N // R)
    return pl.pallas_call(
        functools.partial(_topk_body, nrows, boff),
        grid=grid,
        in_specs=[
            pl.BlockSpec((R, Cp), lambda b, r: (b * (N // R) + r, 0)),
            pl.BlockSpec((1, Cp, N), lambda b, r: (b, 0, 0)),
        ],
        out_specs=pl.BlockSpec((R, IDXW), lambda b, r: (b * (N // R) + r, 0)),
        out_shape=jax.ShapeDtypeStruct((BN, IDXW), jnp.int32),
    )(xrow, xcol)


def _make_gather(nidx, Cp, CH=128):
    per_w = nidx // NW
    npair = per_w // (2 * CH)
    mesh = plsc.VectorSubcoreMesh(core_axis_name="c", subcore_axis_name="s")

    @functools.partial(
        pl.kernel, mesh=mesh,
        out_type=jax.ShapeDtypeStruct((nidx, Cp), jnp.float32),
        scratch_types=[
            pltpu.VMEM((2, CH), jnp.int32),
            pltpu.VMEM((2, CH, Cp), jnp.float32),
            pltpu.SemaphoreType.DMA,
            pltpu.SemaphoreType.DMA,
            pltpu.SemaphoreType.DMA,
            pltpu.SemaphoreType.DMA,
        ],
    )
    def gather(xrow_hbm, idx_hbm, nb_hbm, idx2, rows2, sg0, sg1, sw0, sw1):
        wid = lax.axis_index("s") * NC + lax.axis_index("c")
        base_w = wid * per_w
        sg = (sg0, sg1)
        sw = (sw0, sw1)

        def pair(p, _):
            hs = []
            for b in (0, 1):
                base = base_w + (2 * p + b) * CH

                @pl.when(p > 0)
                def _drain(b=b, base=base):
                    pltpu.make_async_copy(
                        rows2.at[b], nb_hbm.at[pl.ds(base, CH)], sw[b]).wait()

                pltpu.sync_copy(idx_hbm.at[pl.ds(base, CH)], idx2.at[b])
                hs.append(pltpu.async_copy(
                    xrow_hbm.at[idx2.at[b]], rows2.at[b], sg[b]))
            for b in (0, 1):
                base = base_w + (2 * p + b) * CH
                hs[b].wait()
                pltpu.async_copy(rows2.at[b], nb_hbm.at[pl.ds(base, CH)], sw[b])
            return _

        lax.fori_loop(0, npair, pair, 0)
        for b in (0, 1):
            pltpu.make_async_copy(
                rows2.at[b], nb_hbm.at[pl.ds(base_w, CH)], sw[b]).wait()

    return gather


def _conv_body(nb_ref, xc_ref, wa_ref, wb_ref, b_ref, g_ref, be_ref, out_ref):
    xc = xc_ref[...]
    R2, Cp = xc.shape
    O = wa_ref.shape[1]
    nb = nb_ref[...]
    xc_rep = jnp.broadcast_to(xc[:, None, :], (R2, K, Cp)).reshape(R2 * K, Cp)
    d = (nb - xc_rep).astype(jnp.bfloat16)
    ya = jax.lax.dot_general(d, wa_ref[...], (((1,), (0,)), ((), ())),
                             preferred_element_type=jnp.float32)
    ya3 = ya.reshape(R2, K, O)
    m = jnp.max(ya3, axis=1)
    yb = jax.lax.dot_general(xc.astype(jnp.bfloat16), wb_ref[...],
                             (((1,), (0,)), ((), ())),
                             preferred_element_type=jnp.float32)
    y = (m + yb) + b_ref[...]
    y = y / INV_SQRT * g_ref[...] + be_ref[...]
    out_ref[...] = jnp.where(y > 0, y, 0.2 * y)


def _conv(nb, xrow, waT, wbT, bvec, gvec, bevec, R2=128):
    BN, Cp = xrow.shape
    O = waT.shape[1]
    grid = (BN // R2,)
    return pl.pallas_call(
        _conv_body,
        grid=grid,
        in_specs=[
            pl.BlockSpec((R2 * K, Cp), lambda r: (r, 0)),
            pl.BlockSpec((R2, Cp), lambda r: (r, 0)),
            pl.BlockSpec((Cp, O), lambda r: (0, 0)),
            pl.BlockSpec((Cp, O), lambda r: (0, 0)),
            pl.BlockSpec((1, O), lambda r: (0, 0)),
            pl.BlockSpec((1, O), lambda r: (0, 0)),
            pl.BlockSpec((1, O), lambda r: (0, 0)),
        ],
        out_specs=pl.BlockSpec((R2, O), lambda r: (r, 0)),
        out_shape=jax.ShapeDtypeStruct((BN, O), jnp.float32),
    )(nb, xrow, waT, wbT, bvec, gvec, bevec)


def _head_body(x1_ref, x2_ref, x3_ref, x4_ref, w5_refs, b5_ref, g5_ref,
               be5_ref, w6_ref, b6_ref, w7_ref, b7_ref, out_ref):
    w51, w52, w53, w54 = w5_refs
    y5 = jax.lax.dot_general(x1_ref[...].astype(jnp.bfloat16), w51[...],
                             (((1,), (0,)), ((), ())),
                             preferred_element_type=jnp.float32)
    y5 += jax.lax.dot_general(x2_ref[...].astype(jnp.bfloat16), w52[...],
                              (((1,), (0,)), ((), ())),
                              preferred_element_type=jnp.float32)
    y5 += jax.lax.dot_general(x3_ref[...].astype(jnp.bfloat16), w53[...],
                              (((1,), (0,)), ((), ())),
                              preferred_element_type=jnp.float32)
    y5 += jax.lax.dot_general(x4_ref[...].astype(jnp.bfloat16), w54[...],
                              (((1,), (0,)), ((), ())),
                              preferred_element_type=jnp.float32)
    y5 = y5 + b5_ref[...]
    y5 = y5 / INV_SQRT * g5_ref[...] + be5_ref[...]
    y5 = jnp.where(y5 > 0, y5, 0.2 * y5)
    y6 = jax.lax.dot_general(y5.astype(jnp.bfloat16), w6_ref[...],
                             (((1,), (0,)), ((), ())),
                             preferred_element_type=jnp.float32)
    y6 = y6 + b6_ref[...]
    y7 = jax.lax.dot_general(y6.astype(jnp.bfloat16), w7_ref[...],
                             (((1,), (0,)), ((), ())),
                             preferred_element_type=jnp.float32)
    out_ref[...] = y7 + b7_ref[...]


def _head(x1, x2, x3, x4, w5Ts, b5, g5, be5, w6T, b6, w7T, b7, R3=512):
    BN = x1.shape[0]
    grid = (BN // R3,)
    O7 = w7T.shape[1]

    def body(x1r, x2r, x3r, x4r, w51, w52, w53, w54, b5r, g5r, be5r,
             w6r, b6r, w7r, b7r, outr):
        _head_body(x1r, x2r, x3r, x4r, (w51, w52, w53, w54), b5r, g5r,
                   be5r, w6r, b6r, w7r, b7r, outr)

    full = lambda a: pl.BlockSpec(a.shape, lambda r: tuple(0 for _ in a.shape))
    row = lambda a: pl.BlockSpec((R3, a.shape[1]), lambda r: (r, 0))
    args = (x1, x2, x3, x4, *w5Ts, b5, g5, be5, w6T, b6, w7T, b7)
    specs = [row(x1), row(x2), row(x3), row(x4)] + [full(a) for a in args[4:]]
    return pl.pallas_call(
        body,
        grid=grid,
        in_specs=specs,
        out_specs=pl.BlockSpec((R3, O7), lambda r: (r, 0)),
        out_shape=jax.ShapeDtypeStruct((BN, O7), jnp.float32),
    )(*args)


def _pad_cols(a, w):
    if a.shape[-1] == w:
        return a
    return jnp.zeros(a.shape[:-1] + (w,), a.dtype).at[..., :a.shape[-1]].set(a)


def _edge_layer(xrow, xcol, W, b, g, be, C, Cp, O, Op):
    BN = xrow.shape[0]
    B = xcol.shape[0]
    Bh, BNh = B // 2, BN // 2
    waT = jnp.zeros((Cp, Op), jnp.bfloat16).at[:C, :O].set(
        W[:, :C].T.astype(jnp.bfloat16))
    wbT = jnp.zeros((Cp, Op), jnp.bfloat16).at[:C, :O].set(
        W[:, C:].T.astype(jnp.bfloat16))
    bp = _pad_cols(b[None, :], Op)
    gp = _pad_cols(g[None, :], Op)
    bep = _pad_cols(be[None, :], Op)
    gath = _make_gather(BNh * K, Cp)
    idxA = _topk(xrow[:BNh], xcol[:Bh], boff=0)[:, :K].reshape(-1)
    nbA = gath(xrow, idxA)
    idxB = _topk(xrow[BNh:], xcol[Bh:], boff=Bh)[:, :K].reshape(-1)
    xA = _conv(nbA, xrow[:BNh], waT, wbT, bp, gp, bep)
    nbB = gath(xrow, idxB)
    xB = _conv(nbB, xrow[BNh:], waT, wbT, bp, gp, bep)
    return jnp.concatenate([xA, xB], axis=0)


def kernel(x, W1, b1, g1, be1, W2, b2, g2, be2, W3, b3, g3, be3,
           W4, b4, g4, be4, W5, b5, g5, be5, W6, b6, W7, b7):
    B, C0, N = x.shape
    BN = B * N
    xr = jnp.transpose(x, (0, 2, 1)).reshape(BN, C0)
    xrow1 = jnp.zeros((BN, 128), jnp.float32).at[:, :C0].set(xr)
    xcol1 = jnp.zeros((B, 128, N), jnp.float32).at[:, :C0].set(x)

    x1 = _edge_layer(xrow1, xcol1, W1, b1, g1, be1, C0, 128, 64, 128)
    xcol2 = jnp.transpose(x1.reshape(B, N, 128), (0, 2, 1))
    x2 = _edge_layer(x1, xcol2, W2, b2, g2, be2, 64, 128, 64, 128)
    xcol3 = jnp.transpose(x2.reshape(B, N, 128), (0, 2, 1))
    x3 = _edge_layer(x2, xcol3, W3, b3, g3, be3, 64, 128, 128, 128)
    xcol4 = jnp.transpose(x3.reshape(B, N, 128), (0, 2, 1))
    x4 = _edge_layer(x3, xcol4, W4, b4, g4, be4, 128, 128, 256, 256)

    w5Ts = tuple(
        jnp.zeros((128, 1024), jnp.bfloat16).at[:hi - lo].set(
            W5[:, lo:hi].T.astype(jnp.bfloat16))
        for lo, hi in ((0, 64), (64, 128), (128, 256)))
    w5Ts = w5Ts + (W5[:, 256:512].T.astype(jnp.bfloat16),)
    w7T = jnp.zeros((256, 16), jnp.bfloat16).at[:, :13].set(
        W7.T.astype(jnp.bfloat16))
    b7p = jnp.zeros((1, 16), jnp.float32).at[:, :13].set(b7[None, :])
    out = _head(x1, x2, x3, x4, w5Ts, b5[None, :], g5[None, :], be5[None, :],
                W6.T.astype(jnp.bfloat16), b6[None, :], w7T, b7p)
    out = out.reshape(B, N, 16)[:, :, :13]
    return jnp.transpose(out, (0, 2, 1))
